# baseline (speedup 1.0000x reference)
import functools

import jax
import jax.numpy as jnp
from jax import lax
from jax.experimental import pallas as pl
from jax.experimental.pallas import tpu as pltpu
from jax.experimental.pallas import tpu_sc as plsc

B = 16384
D = 64

_info = plsc.get_sparse_core_info()
_NC, _NS = _info.num_cores, _info.num_subcores
NW = _NC * _NS
BPW = B // NW
NG = BPW // 16


def _sc_gather_body(ui, mi, ugt, mgt, umt, mmt,
                    ug_o, mg_o, um_o, mm_o, uiv, miv, sem):
    wid = lax.axis_index("s") * _NC + lax.axis_index("c")
    base = wid * BPW
    pltpu.sync_copy(ui.at[pl.ds(wid * NG, NG)], uiv)
    pltpu.sync_copy(mi.at[pl.ds(wid * NG, NG)], miv)

    for tbl, ivv, out in ((ugt, uiv, ug_o), (mgt, miv, mg_o),
                          (umt, uiv, um_o), (mmt, miv, mm_o)):
        def body(g, carry, tbl=tbl, ivv=ivv, out=out):
            v = ivv[g, :]
            ob = base + g * 16
            for l in range(16):
                pltpu.async_copy(tbl.at[pl.ds(v[l], 1)],
                                 out.at[pl.ds(ob + l, 1)], sem)
            return carry
        lax.fori_loop(0, NG, body, 0)
    for tbl, out in ((ugt, ug_o), (mgt, mg_o), (umt, um_o), (mmt, mm_o)):
        pltpu.make_async_copy(tbl.at[pl.ds(0, BPW)],
                              out.at[pl.ds(base, BPW)], sem).wait()


def _sc_gather(ui, mi, ugt, mgt, umt, mmt):
    mesh = plsc.VectorSubcoreMesh(core_axis_name="c", subcore_axis_name="s")
    f = functools.partial(
        pl.kernel,
        mesh=mesh,
        out_type=[jax.ShapeDtypeStruct((B, D), jnp.float32)] * 4,
        scratch_types=[
            pltpu.VMEM((NG, 16), jnp.int32),
            pltpu.VMEM((NG, 16), jnp.int32),
            pltpu.SemaphoreType.DMA,
        ],
    )(_sc_gather_body)
    return f(ui, mi, ugt, mgt, umt, mmt)


def _tc_dense_body(ug_ref, mg_ref, um_ref, mm_ref, w1u_ref, w1m_ref, b1_ref,
                   w2_ref, b2_ref, wfg_ref, wfm_ref, bf_ref, o_ref):
    um = um_ref[...]
    mm = mm_ref[...]
    h = jnp.maximum(
        jnp.dot(um, w1u_ref[...], preferred_element_type=jnp.float32)
        + jnp.dot(mm, w1m_ref[...], preferred_element_type=jnp.float32)
        + b1_ref[...][None, :], 0.0)
    m = jnp.maximum(
        jnp.dot(h, w2_ref[...], preferred_element_type=jnp.float32)
        + b2_ref[...][None, :], 0.0)
    g = ug_ref[...] * mg_ref[...]
    pred = (jnp.sum(g * wfg_ref[...][None, :], axis=-1)
            + jnp.sum(m * wfm_ref[...][None, :], axis=-1) + bf_ref[0])
    o_ref[...] = pred


def _tc_dense(ug, mg, um, mm, w1u, w1m, b1, w2t, b2, wfg, wfm, bf):
    bb = 2048
    grid = (B // bb,)
    row = lambda i: (i, 0)
    full2 = lambda i: (0, 0)
    full1 = lambda i: (0,)
    return pl.pallas_call(
        _tc_dense_body,
        grid=grid,
        in_specs=[
            pl.BlockSpec((bb, D), row),
            pl.BlockSpec((bb, D), row),
            pl.BlockSpec((bb, D), row),
            pl.BlockSpec((bb, D), row),
            pl.BlockSpec((D, D), full2),
            pl.BlockSpec((D, D), full2),
            pl.BlockSpec((D,), full1),
            pl.BlockSpec((D, D // 2), full2),
            pl.BlockSpec((D // 2,), full1),
            pl.BlockSpec((D,), full1),
            pl.BlockSpec((D // 2,), full1),
            pl.BlockSpec((1,), full1),
        ],
        out_specs=pl.BlockSpec((bb,), lambda i: (i,)),
        out_shape=jax.ShapeDtypeStruct((B,), jnp.float32),
    )(ug, mg, um, mm, w1u, w1m, b1, w2t, b2, wfg, wfm, bf)


def kernel(user_indices, movie_indices, user_gmf_table, movie_gmf_table,
           user_mlp_table, movie_mlp_table, W1, b1, W2, b2, Wf, bf):
    ui = user_indices.astype(jnp.int32).reshape(B // 16, 16)
    mi = movie_indices.astype(jnp.int32).reshape(B // 16, 16)
    ug, mg, um, mm = _sc_gather(ui, mi, user_gmf_table, movie_gmf_table,
                                user_mlp_table, movie_mlp_table)
    w1u = W1[:, :D].T
    w1m = W1[:, D:].T
    w2t = W2.T
    wfg = Wf[0, :D]
    wfm = Wf[0, D:]
    return _tc_dense(ug, mg, um, mm, w1u, w1m, b1, w2t, b2, wfg, wfm, bf)


# speedup vs baseline: 2.2152x; 2.2152x over previous
import functools

import jax
import jax.numpy as jnp
from jax import lax
from jax.experimental import pallas as pl
from jax.experimental.pallas import tpu as pltpu
from jax.experimental.pallas import tpu_sc as plsc

B = 16384
D = 64

_info = plsc.get_sparse_core_info()
_NC, _NS = _info.num_cores, _info.num_subcores
NW = _NC * _NS
BPW = B // NW
NG = BPW // 16


def _sc_gather_body(ui, mi, ugt, mgt, umt, mmt,
                    ug_o, mg_o, um_o, mm_o, uiv, miv, ob0, ob1,
                    gs0, gs1, ws0, ws1):
    wid = lax.axis_index("s") * _NC + lax.axis_index("c")
    base = wid * BPW
    pltpu.sync_copy(ui.at[pl.ds(wid * NG, NG)], uiv)
    pltpu.sync_copy(mi.at[pl.ds(wid * NG, NG)], miv)

    obs = (ob0, ob1)
    gsems = (gs0, gs1)
    wsems = (ws0, ws1)
    CH = BPW // 2
    NGC = CH // 16
    units = []
    for tbl, ivv, out in ((ugt, uiv, ug_o), (mgt, miv, mg_o),
                          (umt, uiv, um_o), (mmt, miv, mm_o)):
        for h in range(2):
            units.append((tbl, ivv, out, h))
    nu = len(units)
    for k in range(nu + 1):
        if k < nu:
            tbl, ivv, out, h = units[k]
            b = k % 2
            if k >= 2:
                _, _, pout, ph = units[k - 2]
                pltpu.make_async_copy(
                    obs[b], pout.at[pl.ds(base, CH)], wsems[b]).wait()

            def body(g, carry, tbl=tbl, ivv=ivv, h=h, b=b):
                v = ivv[h * NGC + g, :]
                ob = g * 16
                for l in range(16):
                    pltpu.async_copy(tbl.at[pl.ds(v[l], 1)],
                                     obs[b].at[pl.ds(ob + l, 1)], gsems[b])
                return carry
            lax.fori_loop(0, NGC, body, 0)
        j = k - 1
        if j >= 0:
            tbl, ivv, out, h = units[j]
            bj = j % 2
            pltpu.make_async_copy(tbl.at[pl.ds(0, CH)], obs[bj],
                                  gsems[bj]).wait()
            pltpu.async_copy(obs[bj],
                             out.at[pl.ds(base + h * CH, CH)], wsems[bj])
    out = units[nu - 1][2]
    pltpu.make_async_copy(obs[1], out.at[pl.ds(base, CH)], wsems[1]).wait()
    pltpu.make_async_copy(obs[0], out.at[pl.ds(base, CH)], wsems[0]).wait()


def _sc_gather(ui, mi, ugt, mgt, umt, mmt):
    mesh = plsc.VectorSubcoreMesh(core_axis_name="c", subcore_axis_name="s")
    f = functools.partial(
        pl.kernel,
        mesh=mesh,
        out_type=[jax.ShapeDtypeStruct((B, D), jnp.float32)] * 4,
        scratch_types=[
            pltpu.VMEM((NG, 16), jnp.int32),
            pltpu.VMEM((NG, 16), jnp.int32),
            pltpu.VMEM((BPW // 2, D), jnp.float32),
            pltpu.VMEM((BPW // 2, D), jnp.float32),
            pltpu.SemaphoreType.DMA,
            pltpu.SemaphoreType.DMA,
            pltpu.SemaphoreType.DMA,
            pltpu.SemaphoreType.DMA,
        ],
    )(_sc_gather_body)
    return f(ui, mi, ugt, mgt, umt, mmt)


def _tc_dense_body(ug_ref, mg_ref, um_ref, mm_ref, w1u_ref, w1m_ref, b1_ref,
                   w2_ref, b2_ref, wfg_ref, wfm_ref, bf_ref, o_ref):
    um = um_ref[...]
    mm = mm_ref[...]
    h = jnp.maximum(
        jnp.dot(um, w1u_ref[...], preferred_element_type=jnp.float32)
        + jnp.dot(mm, w1m_ref[...], preferred_element_type=jnp.float32)
        + b1_ref[...][None, :], 0.0)
    m = jnp.maximum(
        jnp.dot(h, w2_ref[...], preferred_element_type=jnp.float32)
        + b2_ref[...][None, :], 0.0)
    g = ug_ref[...] * mg_ref[...]
    pred = (jnp.sum(g * wfg_ref[...][None, :], axis=-1)
            + jnp.sum(m * wfm_ref[...][None, :], axis=-1) + bf_ref[0])
    o_ref[...] = pred


def _tc_dense(ug, mg, um, mm, w1u, w1m, b1, w2t, b2, wfg, wfm, bf):
    bb = 2048
    grid = (B // bb,)
    row = lambda i: (i, 0)
    full2 = lambda i: (0, 0)
    full1 = lambda i: (0,)
    return pl.pallas_call(
        _tc_dense_body,
        grid=grid,
        in_specs=[
            pl.BlockSpec((bb, D), row),
            pl.BlockSpec((bb, D), row),
            pl.BlockSpec((bb, D), row),
            pl.BlockSpec((bb, D), row),
            pl.BlockSpec((D, D), full2),
            pl.BlockSpec((D, D), full2),
            pl.BlockSpec((D,), full1),
            pl.BlockSpec((D, D // 2), full2),
            pl.BlockSpec((D // 2,), full1),
            pl.BlockSpec((D,), full1),
            pl.BlockSpec((D // 2,), full1),
            pl.BlockSpec((1,), full1),
        ],
        out_specs=pl.BlockSpec((bb,), lambda i: (i,)),
        out_shape=jax.ShapeDtypeStruct((B,), jnp.float32),
    )(ug, mg, um, mm, w1u, w1m, b1, w2t, b2, wfg, wfm, bf)


def kernel(user_indices, movie_indices, user_gmf_table, movie_gmf_table,
           user_mlp_table, movie_mlp_table, W1, b1, W2, b2, Wf, bf):
    ui = user_indices.astype(jnp.int32).reshape(B // 16, 16)
    mi = movie_indices.astype(jnp.int32).reshape(B // 16, 16)
    ug, mg, um, mm = _sc_gather(ui, mi, user_gmf_table, movie_gmf_table,
                                user_mlp_table, movie_mlp_table)
    w1u = W1[:, :D].T
    w1m = W1[:, D:].T
    w2t = W2.T
    wfg = Wf[0, :D]
    wfm = Wf[0, D:]
    return _tc_dense(ug, mg, um, mm, w1u, w1m, b1, w2t, b2, wfg, wfm, bf)


# speedup vs baseline: 2.2157x; 1.0002x over previous
import functools

import jax
import jax.numpy as jnp
from jax import lax
from jax.experimental import pallas as pl
from jax.experimental.pallas import tpu as pltpu
from jax.experimental.pallas import tpu_sc as plsc

B = 16384
D = 64

_info = plsc.get_sparse_core_info()
_NC, _NS = _info.num_cores, _info.num_subcores
NW = _NC * _NS
BPW = B // NW
NG = BPW // 16


def _sc_gather_body(ui, mi, ugt, mgt, umt, mmt,
                    ug_o, mg_o, um_o, mm_o, uiv, miv, ob0, ob1,
                    gs0, gs1, gs2, gs3, gs4, gs5, gs6, gs7, ws0, ws1):
    wid = lax.axis_index("s") * _NC + lax.axis_index("c")
    base = wid * BPW
    pltpu.sync_copy(ui.at[pl.ds(wid * NG, NG)], uiv)
    pltpu.sync_copy(mi.at[pl.ds(wid * NG, NG)], miv)

    obs = (ob0, ob1)
    gsems = ((gs0, gs1, gs2, gs3), (gs4, gs5, gs6, gs7))
    wsems = (ws0, ws1)
    CH = BPW // 2
    NGC = CH // 16
    units = []
    for tbl, ivv, out in ((ugt, uiv, ug_o), (mgt, miv, mg_o),
                          (umt, uiv, um_o), (mmt, miv, mm_o)):
        for h in range(2):
            units.append((tbl, ivv, out, h))
    nu = len(units)
    for k in range(nu + 1):
        if k < nu:
            tbl, ivv, out, h = units[k]
            b = k % 2
            if k >= 2:
                _, _, pout, ph = units[k - 2]
                pltpu.make_async_copy(
                    obs[b], pout.at[pl.ds(base, CH)], wsems[b]).wait()

            def body(g, carry, tbl=tbl, ivv=ivv, h=h, b=b):
                v = ivv[h * NGC + g, :]
                ob = g * 16
                for l in range(16):
                    pltpu.async_copy(tbl.at[pl.ds(v[l], 1)],
                                     obs[b].at[pl.ds(ob + l, 1)],
                                     gsems[b][l % 4])
                return carry
            lax.fori_loop(0, NGC, body, 0)
        j = k - 1
        if j >= 0:
            tbl, ivv, out, h = units[j]
            bj = j % 2
            for q in range(4):
                pltpu.make_async_copy(tbl.at[pl.ds(0, CH // 4)],
                                      obs[bj].at[pl.ds(0, CH // 4)],
                                      gsems[bj][q]).wait()
            pltpu.async_copy(obs[bj],
                             out.at[pl.ds(base + h * CH, CH)], wsems[bj])
    out = units[nu - 1][2]
    pltpu.make_async_copy(obs[1], out.at[pl.ds(base, CH)], wsems[1]).wait()
    pltpu.make_async_copy(obs[0], out.at[pl.ds(base, CH)], wsems[0]).wait()


def _sc_gather(ui, mi, ugt, mgt, umt, mmt):
    mesh = plsc.VectorSubcoreMesh(core_axis_name="c", subcore_axis_name="s")
    f = functools.partial(
        pl.kernel,
        mesh=mesh,
        out_type=[jax.ShapeDtypeStruct((B, D), jnp.float32)] * 4,
        scratch_types=[
            pltpu.VMEM((NG, 16), jnp.int32),
            pltpu.VMEM((NG, 16), jnp.int32),
            pltpu.VMEM((BPW // 2, D), jnp.float32),
            pltpu.VMEM((BPW // 2, D), jnp.float32),
        ] + [pltpu.SemaphoreType.DMA] * 10,
    )(_sc_gather_body)
    return f(ui, mi, ugt, mgt, umt, mmt)


def _tc_dense_body(ug_ref, mg_ref, um_ref, mm_ref, w1u_ref, w1m_ref, b1_ref,
                   w2_ref, b2_ref, wfg_ref, wfm_ref, bf_ref, o_ref):
    um = um_ref[...]
    mm = mm_ref[...]
    h = jnp.maximum(
        jnp.dot(um, w1u_ref[...], preferred_element_type=jnp.float32)
        + jnp.dot(mm, w1m_ref[...], preferred_element_type=jnp.float32)
        + b1_ref[...][None, :], 0.0)
    m = jnp.maximum(
        jnp.dot(h, w2_ref[...], preferred_element_type=jnp.float32)
        + b2_ref[...][None, :], 0.0)
    g = ug_ref[...] * mg_ref[...]
    pred = (jnp.sum(g * wfg_ref[...][None, :], axis=-1)
            + jnp.sum(m * wfm_ref[...][None, :], axis=-1) + bf_ref[0])
    o_ref[...] = pred


def _tc_dense(ug, mg, um, mm, w1u, w1m, b1, w2t, b2, wfg, wfm, bf):
    bb = 2048
    grid = (B // bb,)
    row = lambda i: (i, 0)
    full2 = lambda i: (0, 0)
    full1 = lambda i: (0,)
    return pl.pallas_call(
        _tc_dense_body,
        grid=grid,
        in_specs=[
            pl.BlockSpec((bb, D), row),
            pl.BlockSpec((bb, D), row),
            pl.BlockSpec((bb, D), row),
            pl.BlockSpec((bb, D), row),
            pl.BlockSpec((D, D), full2),
            pl.BlockSpec((D, D), full2),
            pl.BlockSpec((D,), full1),
            pl.BlockSpec((D, D // 2), full2),
            pl.BlockSpec((D // 2,), full1),
            pl.BlockSpec((D,), full1),
            pl.BlockSpec((D // 2,), full1),
            pl.BlockSpec((1,), full1),
        ],
        out_specs=pl.BlockSpec((bb,), lambda i: (i,)),
        out_shape=jax.ShapeDtypeStruct((B,), jnp.float32),
    )(ug, mg, um, mm, w1u, w1m, b1, w2t, b2, wfg, wfm, bf)


def kernel(user_indices, movie_indices, user_gmf_table, movie_gmf_table,
           user_mlp_table, movie_mlp_table, W1, b1, W2, b2, Wf, bf):
    ui = user_indices.astype(jnp.int32).reshape(B // 16, 16)
    mi = movie_indices.astype(jnp.int32).reshape(B // 16, 16)
    ug, mg, um, mm = _sc_gather(ui, mi, user_gmf_table, movie_gmf_table,
                                user_mlp_table, movie_mlp_table)
    w1u = W1[:, :D].T
    w1m = W1[:, D:].T
    w2t = W2.T
    wfg = Wf[0, :D]
    wfm = Wf[0, D:]
    return _tc_dense(ug, mg, um, mm, w1u, w1m, b1, w2t, b2, wfg, wfm, bf)
